# baseline (speedup 1.0000x reference)
import functools

import jax
import jax.numpy as jnp
from jax import lax
from jax.experimental import pallas as pl
from jax.experimental.pallas import tpu as pltpu
from jax.experimental.pallas import tpu_sc as plsc

B = 4096
NUM_NUM = 13
N_CAT = 26
VOCAB = 100000
D = 64
NTOK = NUM_NUM + N_CAT

NCORES = 2
NSUB = 16
LANES = 16
NW = NCORES * NSUB
BPW = B // NW
NBUF = 2
SUB = 8
PROWS = N_CAT * SUB
XN_PAD = 16
NROWS = N_CAT * VOCAB
HI0 = N_CAT - LANES

_mesh = plsc.VectorSubcoreMesh(core_axis_name="c", subcore_axis_name="s")


@functools.partial(
    pl.kernel,
    out_type=jax.ShapeDtypeStruct((B, NTOK, D), jnp.float32),
    mesh=_mesh,
    scratch_types=[
        pltpu.VMEM((BPW, N_CAT), jnp.int32),
        pltpu.VMEM((BPW, XN_PAD), jnp.float32),
        pltpu.VMEM((NUM_NUM, D), jnp.float32),
        pltpu.VMEM((NUM_NUM, D), jnp.float32),
        pltpu.VMEM((PROWS, D), jnp.float32),
        pltpu.VMEM((PROWS, D), jnp.float32),
        pltpu.VMEM((1, NTOK, D), jnp.float32),
        pltpu.VMEM((1, NTOK, D), jnp.float32),
        pltpu.SemaphoreType.DMA,
        pltpu.SemaphoreType.DMA,
        pltpu.SemaphoreType.DMA,
        pltpu.SemaphoreType.DMA,
    ],
)
def _tokenizer(xnum_hbm, xcat_hbm, a_hbm, b_hbm, tab_hbm, out_hbm,
               xc_v, xn_v, a_v, b_v,
               pbuf0, pbuf1, cbuf0, cbuf1, gsem0, gsem1, osem0, osem1):
    pbufs = [pbuf0, pbuf1]
    cbufs = [cbuf0, cbuf1]
    gsems = [gsem0, gsem1]
    osems = [osem0, osem1]

    wid = lax.axis_index("s") * NCORES + lax.axis_index("c")
    b0 = wid * BPW

    pltpu.sync_copy(xcat_hbm.at[pl.ds(b0, BPW)], xc_v)
    pltpu.sync_copy(xnum_hbm.at[pl.ds(b0, BPW)], xn_v)
    pltpu.sync_copy(a_hbm, a_v)
    pltpu.sync_copy(b_hbm, b_v)

    def _issue_fetches(c, s):
        plo = xc_v[c, pl.ds(0, LANES)]
        phi = xc_v[c, pl.ds(HI0, LANES)]
        for j in range(N_CAT):
            xcj = plo[j] if j < LANES else phi[j - HI0]
            g = lax.shift_right_logical(xcj, 3)
            row0 = pl.multiple_of(g * SUB, SUB)
            pltpu.async_copy(
                tab_hbm.at[j, pl.ds(row0, SUB)],
                pbufs[s].at[pl.ds(j * SUB, SUB)],
                gsems[s])

    def _drain_fetches(s):
        pltpu.make_async_copy(
            tab_hbm.at[0, pl.ds(0, PROWS)], pbufs[s], gsems[s]).wait()

    def _out_descr(c, s):
        return pltpu.make_async_copy(
            cbufs[s], out_hbm.at[pl.ds(b0 + c, 1)], osems[s])

    _issue_fetches(0, 0)

    def _super_body(g, _):
        for s in range(NBUF):
            c = g * NBUF + s
            s1 = 1 - s

            @pl.when(c >= NBUF)
            def _():
                _out_descr(c - NBUF, s).wait()

            xrow = xn_v[c, pl.ds(0, LANES)]
            for k in range(NUM_NUM):
                x = xrow[k]
                for d0 in range(D // LANES):
                    cbufs[s][0, k, pl.ds(d0 * LANES, LANES)] = (
                        a_v[k, pl.ds(d0 * LANES, LANES)]
                        + b_v[k, pl.ds(d0 * LANES, LANES)] * x)

            _drain_fetches(s)
            plo = xc_v[c, pl.ds(0, LANES)]
            phi = xc_v[c, pl.ds(HI0, LANES)]
            for j in range(N_CAT):
                sub = (plo[j] if j < LANES else phi[j - HI0]) & (SUB - 1)
                for d0 in range(D // LANES):
                    cbufs[s][0, NUM_NUM + j, pl.ds(d0 * LANES, LANES)] = (
                        pbufs[s][j * SUB + sub, pl.ds(d0 * LANES, LANES)])

            _out_descr(c, s).start()

            @pl.when(c + 1 < BPW)
            def _():
                _issue_fetches(c + 1, s1)
        return 0

    lax.fori_loop(0, BPW // NBUF, _super_body, 0)

    for s in range(NBUF):
        _out_descr(BPW - NBUF + s, s).wait()


def kernel(x_num, x_cat, a, b, tables):
    xn = jnp.pad(x_num, ((0, 0), (0, XN_PAD - NUM_NUM)))
    return _tokenizer(xn, x_cat, a, b, tables)


# speedup vs baseline: 1.6594x; 1.6594x over previous
import functools

import jax
import jax.numpy as jnp
from jax import lax
from jax.experimental import pallas as pl
from jax.experimental.pallas import tpu as pltpu
from jax.experimental.pallas import tpu_sc as plsc

B = 4096
NUM_NUM = 13
N_CAT = 26
VOCAB = 100000
D = 64
NTOK = NUM_NUM + N_CAT

NCORES = 2
NSUB = 16
LANES = 16
NW = NCORES * NSUB
BPW = B // NW
NBUF = 2
SUB = 8
PROWS = N_CAT * SUB
XN_PAD = 16
NROWS = N_CAT * VOCAB
HI0 = N_CAT - LANES

_mesh = plsc.VectorSubcoreMesh(core_axis_name="c", subcore_axis_name="s")


@functools.partial(
    pl.kernel,
    out_type=jax.ShapeDtypeStruct((B, NTOK, D), jnp.float32),
    mesh=_mesh,
    scratch_types=[
        pltpu.VMEM((BPW, N_CAT), jnp.int32),
        pltpu.VMEM((BPW, XN_PAD), jnp.float32),
        pltpu.VMEM((NUM_NUM, D), jnp.float32),
        pltpu.VMEM((NUM_NUM, D), jnp.float32),
        pltpu.VMEM((PROWS, D), jnp.float32),
        pltpu.VMEM((PROWS, D), jnp.float32),
        pltpu.VMEM((1, NTOK, D), jnp.float32),
        pltpu.VMEM((1, NTOK, D), jnp.float32),
        pltpu.SemaphoreType.DMA,
        pltpu.SemaphoreType.DMA,
        pltpu.SemaphoreType.DMA,
        pltpu.SemaphoreType.DMA,
    ],
)
def _tokenizer(xnum_hbm, xcat_hbm, a_hbm, b_hbm, tab_hbm, out_hbm,
               xc_v, xn_v, a_v, b_v,
               pbuf0, pbuf1, cbuf0, cbuf1, gsem0, gsem1, osem0, osem1):
    pbufs = [pbuf0, pbuf1]
    cbufs = [cbuf0, cbuf1]
    gsems = [gsem0, gsem1]
    osems = [osem0, osem1]

    wid = lax.axis_index("s") * NCORES + lax.axis_index("c")
    b0 = wid * BPW

    pltpu.sync_copy(xcat_hbm.at[pl.ds(b0, BPW)], xc_v)
    pltpu.sync_copy(xnum_hbm.at[pl.ds(b0, BPW)], xn_v)
    pltpu.sync_copy(a_hbm, a_v)
    pltpu.sync_copy(b_hbm, b_v)

    def _issue_fetches(c, s):
        plo = xc_v[c, pl.ds(0, LANES)]
        phi = xc_v[c, pl.ds(HI0, LANES)]
        for j in range(N_CAT):
            xcj = plo[j] if j < LANES else phi[j - HI0]
            g = lax.shift_right_logical(xcj + j * VOCAB, 3)
            row0 = pl.multiple_of(g * SUB, SUB)
            pltpu.async_copy(
                tab_hbm.at[pl.ds(row0, SUB)],
                pbufs[s].at[pl.ds(j * SUB, SUB)],
                gsems[s])

    def _drain_fetches(s):
        pltpu.make_async_copy(
            tab_hbm.at[pl.ds(0, PROWS)], pbufs[s], gsems[s]).wait()

    def _out_descr(c, s):
        return pltpu.make_async_copy(
            cbufs[s], out_hbm.at[pl.ds(b0 + c, 1)], osems[s])

    _issue_fetches(0, 0)

    def _super_body(g, _):
        for s in range(NBUF):
            c = g * NBUF + s
            s1 = 1 - s

            @pl.when(c >= NBUF)
            def _():
                _out_descr(c - NBUF, s).wait()

            @pl.when(c + 1 < BPW)
            def _():
                _issue_fetches(c + 1, s1)

            xrow = xn_v[c, pl.ds(0, LANES)]
            for k in range(NUM_NUM):
                x = xrow[k]
                for d0 in range(D // LANES):
                    cbufs[s][0, k, pl.ds(d0 * LANES, LANES)] = (
                        a_v[k, pl.ds(d0 * LANES, LANES)]
                        + b_v[k, pl.ds(d0 * LANES, LANES)] * x)

            _drain_fetches(s)
            plo = xc_v[c, pl.ds(0, LANES)]
            phi = xc_v[c, pl.ds(HI0, LANES)]
            for j in range(N_CAT):
                sub = (plo[j] if j < LANES else phi[j - HI0]) & (SUB - 1)
                for d0 in range(D // LANES):
                    cbufs[s][0, NUM_NUM + j, pl.ds(d0 * LANES, LANES)] = (
                        pbufs[s][j * SUB + sub, pl.ds(d0 * LANES, LANES)])

            _out_descr(c, s).start()
        return 0

    lax.fori_loop(0, BPW // NBUF, _super_body, 0)

    for s in range(NBUF):
        _out_descr(BPW - NBUF + s, s).wait()


def kernel(x_num, x_cat, a, b, tables):
    xn = jnp.pad(x_num, ((0, 0), (0, XN_PAD - NUM_NUM)))
    tab = tables.reshape(NROWS, D)
    return _tokenizer(xn, x_cat, a, b, tab)
